# baseline (device time: 108824 ns/iter reference)
import jax
import jax.numpy as jnp
from jax import lax
from jax.experimental import pallas as pl
from jax.experimental.pallas import tpu as pltpu

N_DEV = 16
E_PER = 8
N_EXP = 128
N_TOK = 2048
H = 1024
HH = H // 2
CHUNK = N_TOK // N_DEV
N_STEP = N_DEV - 1

RING = (0, 4, 8, 12, 15, 11, 7, 3, 2, 6, 10, 14, 13, 9, 5, 1)


def kernel(x, router_W, route_idx, expert_W, shared_W):
    ring = jnp.asarray(RING, dtype=jnp.int32)
    my = lax.axis_index("i")
    r = jnp.argmax(ring == my).astype(jnp.int32)
    idx = jnp.arange(N_DEV, dtype=jnp.int32)
    right = ring[jnp.mod(r + 1, N_DEV)].reshape(1)
    left = ring[jnp.mod(r - 1, N_DEV)].reshape(1)
    fwd_sched = ring[jnp.mod(r - 1 - idx, N_DEV)]
    bwd_sched = ring[jnp.mod(r + 1 + idx, N_DEV)]

    def body(x_ref, router_ref, route_ref, w_ref, shared_ref,
             right_ref, left_ref, fsched_ref, bsched_ref, out_ref,
             psel_ref, commf_ref, commb_ref, accf_ref, accb_ref,
             sendf_sems, recvf_sems, sendb_sems, recvb_sems):
        me = lax.axis_index("i")
        rgt = right_ref[0]
        lft = left_ref[0]

        scores = jnp.dot(x_ref[:, :], router_ref[:, :],
                         preferred_element_type=jnp.float32)
        smax = jnp.max(scores, axis=1, keepdims=True)
        p = jnp.exp(scores - smax)
        denom = jnp.sum(p, axis=1, keepdims=True)
        cols = lax.broadcasted_iota(jnp.int32, (N_TOK, N_EXP), 1)
        psel_ref[:, :] = jnp.sum(jnp.where(cols == route_ref[:, :], p, 0.0),
                                 axis=1, keepdims=True) / denom

        def local_contrib(c, lo, acc_ref):
            rows = pl.ds(c * CHUNK, CHUNK)
            xb = x_ref[rows, :]
            rb = route_ref[rows, :]
            pb = psel_ref[rows, :]
            acc_ref[:, :] = jnp.zeros((CHUNK, HH), jnp.float32)
            for ei in range(E_PER):
                ge = me * E_PER + ei
                coeff = jnp.where(rb == ge, pb, 0.0)

                @pl.when(jnp.sum(coeff) > 0.0)
                def _():
                    acc_ref[:, :] = acc_ref[:, :] + jnp.dot(
                        xb * coeff, w_ref[ei, :, lo:lo + HH],
                        preferred_element_type=jnp.float32)

        own_rows = pl.ds(me * CHUNK, CHUNK)
        xown = x_ref[own_rows, :]
        shared_out = jnp.dot(xown, shared_ref[:, :],
                             preferred_element_type=jnp.float32)

        local_contrib(fsched_ref[0], 0, accf_ref)
        local_contrib(bsched_ref[0], HH, accb_ref)
        commf_ref[0] = accf_ref[:, :].astype(jnp.bfloat16)
        commb_ref[0] = accb_ref[:, :].astype(jnp.bfloat16)

        barrier = pltpu.get_barrier_semaphore()
        for nbr in (lft, rgt):
            pl.semaphore_signal(barrier, inc=1, device_id=(nbr,),
                                device_id_type=pl.DeviceIdType.MESH)
        pl.semaphore_wait(barrier, 2)

        for s in range(N_STEP):
            rdma_f = pltpu.make_async_remote_copy(
                src_ref=commf_ref.at[s],
                dst_ref=commf_ref.at[s + 1],
                send_sem=sendf_sems.at[s],
                recv_sem=recvf_sems.at[s],
                device_id=(rgt,),
                device_id_type=pl.DeviceIdType.MESH,
            )
            rdma_b = pltpu.make_async_remote_copy(
                src_ref=commb_ref.at[s],
                dst_ref=commb_ref.at[s + 1],
                send_sem=sendb_sems.at[s],
                recv_sem=recvb_sems.at[s],
                device_id=(lft,),
                device_id_type=pl.DeviceIdType.MESH,
            )
            rdma_f.start()
            rdma_b.start()
            local_contrib(fsched_ref[s + 1], 0, accf_ref)
            local_contrib(bsched_ref[s + 1], HH, accb_ref)
            rdma_f.wait()
            rdma_b.wait()
            commf_ref[s + 1] = (
                commf_ref[s + 1].astype(jnp.float32) + accf_ref[:, :]
            ).astype(jnp.bfloat16)
            commb_ref[s + 1] = (
                commb_ref[s + 1].astype(jnp.float32) + accb_ref[:, :]
            ).astype(jnp.bfloat16)

        out_ref[:, 0:HH] = (
            commf_ref[N_STEP].astype(jnp.float32) + shared_out[:, 0:HH])
        out_ref[:, HH:H] = (
            commb_ref[N_STEP].astype(jnp.float32) + shared_out[:, HH:H])

    return pl.pallas_call(
        body,
        out_shape=jax.ShapeDtypeStruct((CHUNK, H), jnp.float32),
        in_specs=(
            [pl.BlockSpec(memory_space=pltpu.VMEM)] * 5
            + [pl.BlockSpec(memory_space=pltpu.SMEM)] * 4
        ),
        out_specs=pl.BlockSpec(memory_space=pltpu.VMEM),
        scratch_shapes=[
            pltpu.VMEM((N_TOK, 1), jnp.float32),
            pltpu.VMEM((N_DEV, CHUNK, HH), jnp.bfloat16),
            pltpu.VMEM((N_DEV, CHUNK, HH), jnp.bfloat16),
            pltpu.VMEM((CHUNK, HH), jnp.float32),
            pltpu.VMEM((CHUNK, HH), jnp.float32),
            pltpu.SemaphoreType.DMA((N_STEP,)),
            pltpu.SemaphoreType.DMA((N_STEP,)),
            pltpu.SemaphoreType.DMA((N_STEP,)),
            pltpu.SemaphoreType.DMA((N_STEP,)),
        ],
        compiler_params=pltpu.CompilerParams(collective_id=0),
    )(x, router_W, route_idx, expert_W, shared_W,
      right, left, fwd_sched, bwd_sched)


# device time: 73686 ns/iter; 1.4769x vs baseline; 1.4769x over previous
import jax
import jax.numpy as jnp
from jax import lax
from jax.experimental import pallas as pl
from jax.experimental.pallas import tpu as pltpu

N_DEV = 16
E_PER = 8
N_EXP = 128
N_TOK = 2048
H = 1024
HH = H // 2
CHUNK = N_TOK // N_DEV
N_STEP = N_DEV - 1

RING = (0, 4, 8, 12, 15, 11, 7, 3, 2, 6, 10, 14, 13, 9, 5, 1)


def kernel(x, router_W, route_idx, expert_W, shared_W):
    ring = jnp.asarray(RING, dtype=jnp.int32)
    my = lax.axis_index("i")
    r = jnp.argmax(ring == my).astype(jnp.int32)
    idx = jnp.arange(N_DEV, dtype=jnp.int32)
    right = ring[jnp.mod(r + 1, N_DEV)].reshape(1)
    left = ring[jnp.mod(r - 1, N_DEV)].reshape(1)
    fwd_sched = ring[jnp.mod(r - 1 - idx, N_DEV)]
    bwd_sched = ring[jnp.mod(r + 1 + idx, N_DEV)]

    def body(x_ref, router_ref, route_ref, w_ref, shared_ref,
             right_ref, left_ref, fsched_ref, bsched_ref, out_ref,
             psel_ref, commf_ref, commb_ref,
             sendf_sems, recvf_sems, sendb_sems, recvb_sems):
        me = lax.axis_index("i")
        rgt = right_ref[0]
        lft = left_ref[0]

        scores = jnp.dot(x_ref[:, :], router_ref[:, :],
                         preferred_element_type=jnp.float32)
        smax = jnp.max(scores, axis=1, keepdims=True)
        p = jnp.exp(scores - smax)
        denom = jnp.sum(p, axis=1, keepdims=True)
        cols = lax.broadcasted_iota(jnp.int32, (N_TOK, N_EXP), 1)
        psel_ref[:, :] = jnp.sum(jnp.where(cols == route_ref[:, :], p, 0.0),
                                 axis=1, keepdims=True) / denom

        def local_contrib(c, lo):
            rows = pl.ds(c * CHUNK, CHUNK)
            xb = x_ref[rows, :]
            rb = route_ref[rows, :]
            pb = psel_ref[rows, :]
            acc = jnp.zeros((CHUNK, HH), jnp.float32)
            for ei in range(E_PER):
                ge = me * E_PER + ei
                coeff = jnp.where(rb == ge, pb, 0.0)
                acc = acc + jnp.dot(xb * coeff, w_ref[ei, :, lo:lo + HH],
                                    preferred_element_type=jnp.float32)
            return acc

        own_rows = pl.ds(me * CHUNK, CHUNK)
        xown = x_ref[own_rows, :]
        shared_out = jnp.dot(xown, shared_ref[:, :],
                             preferred_element_type=jnp.float32)

        commf_ref[0] = local_contrib(fsched_ref[0], 0).astype(jnp.bfloat16)
        commb_ref[0] = local_contrib(bsched_ref[0], HH).astype(jnp.bfloat16)

        barrier = pltpu.get_barrier_semaphore()
        for nbr in (lft, rgt):
            pl.semaphore_signal(barrier, inc=1, device_id=(nbr,),
                                device_id_type=pl.DeviceIdType.MESH)
        pl.semaphore_wait(barrier, 2)

        for s in range(N_STEP):
            rdma_f = pltpu.make_async_remote_copy(
                src_ref=commf_ref.at[s],
                dst_ref=commf_ref.at[s + 1],
                send_sem=sendf_sems.at[s],
                recv_sem=recvf_sems.at[s],
                device_id=(rgt,),
                device_id_type=pl.DeviceIdType.MESH,
            )
            rdma_b = pltpu.make_async_remote_copy(
                src_ref=commb_ref.at[s],
                dst_ref=commb_ref.at[s + 1],
                send_sem=sendb_sems.at[s],
                recv_sem=recvb_sems.at[s],
                device_id=(lft,),
                device_id_type=pl.DeviceIdType.MESH,
            )
            rdma_f.start()
            rdma_b.start()
            contrib_f = local_contrib(fsched_ref[s + 1], 0)
            contrib_b = local_contrib(bsched_ref[s + 1], HH)
            rdma_f.wait()
            rdma_b.wait()
            commf_ref[s + 1] = (
                commf_ref[s + 1].astype(jnp.float32) + contrib_f
            ).astype(jnp.bfloat16)
            commb_ref[s + 1] = (
                commb_ref[s + 1].astype(jnp.float32) + contrib_b
            ).astype(jnp.bfloat16)

        out_ref[:, 0:HH] = (
            commf_ref[N_STEP].astype(jnp.float32) + shared_out[:, 0:HH])
        out_ref[:, HH:H] = (
            commb_ref[N_STEP].astype(jnp.float32) + shared_out[:, HH:H])

    return pl.pallas_call(
        body,
        out_shape=jax.ShapeDtypeStruct((CHUNK, H), jnp.float32),
        in_specs=(
            [pl.BlockSpec(memory_space=pltpu.VMEM)] * 5
            + [pl.BlockSpec(memory_space=pltpu.SMEM)] * 4
        ),
        out_specs=pl.BlockSpec(memory_space=pltpu.VMEM),
        scratch_shapes=[
            pltpu.VMEM((N_TOK, 1), jnp.float32),
            pltpu.VMEM((N_DEV, CHUNK, HH), jnp.bfloat16),
            pltpu.VMEM((N_DEV, CHUNK, HH), jnp.bfloat16),
            pltpu.SemaphoreType.DMA((N_STEP,)),
            pltpu.SemaphoreType.DMA((N_STEP,)),
            pltpu.SemaphoreType.DMA((N_STEP,)),
            pltpu.SemaphoreType.DMA((N_STEP,)),
        ],
        compiler_params=pltpu.CompilerParams(collective_id=0),
    )(x, router_W, route_idx, expert_W, shared_W,
      right, left, fwd_sched, bwd_sched)


# device time: 71027 ns/iter; 1.5321x vs baseline; 1.0374x over previous
import jax
import jax.numpy as jnp
from jax import lax
from jax.experimental import pallas as pl
from jax.experimental.pallas import tpu as pltpu

N_DEV = 16
E_PER = 8
N_EXP = 128
N_TOK = 2048
H = 1024
HH = H // 2
CHUNK = N_TOK // N_DEV
N_STEP = N_DEV - 1

RING = (0, 4, 8, 12, 15, 11, 7, 3, 2, 6, 10, 14, 13, 9, 5, 1)


def kernel(x, router_W, route_idx, expert_W, shared_W):
    expert_W = expert_W.astype(jnp.bfloat16)
    ring = jnp.asarray(RING, dtype=jnp.int32)
    my = lax.axis_index("i")
    r = jnp.argmax(ring == my).astype(jnp.int32)
    idx = jnp.arange(N_DEV, dtype=jnp.int32)
    right = ring[jnp.mod(r + 1, N_DEV)].reshape(1)
    left = ring[jnp.mod(r - 1, N_DEV)].reshape(1)
    fwd_sched = ring[jnp.mod(r - 1 - idx, N_DEV)]
    bwd_sched = ring[jnp.mod(r + 1 + idx, N_DEV)]

    def body(x_ref, router_ref, route_ref, w_ref, shared_ref,
             right_ref, left_ref, fsched_ref, bsched_ref, out_ref,
             psel_ref, commf_ref, commb_ref,
             sendf_sems, recvf_sems, sendb_sems, recvb_sems):
        me = lax.axis_index("i")
        rgt = right_ref[0]
        lft = left_ref[0]

        scores = jnp.dot(x_ref[:, :], router_ref[:, :],
                         preferred_element_type=jnp.float32)
        smax = jnp.max(scores, axis=1, keepdims=True)
        p = jnp.exp(scores - smax)
        denom = jnp.sum(p, axis=1, keepdims=True)
        cols = lax.broadcasted_iota(jnp.int32, (N_TOK, N_EXP), 1)
        psel_ref[:, :] = jnp.sum(jnp.where(cols == route_ref[:, :], p, 0.0),
                                 axis=1, keepdims=True) / denom

        def local_contrib(c, lo):
            rows = pl.ds(c * CHUNK, CHUNK)
            xb = x_ref[rows, :]
            rb = route_ref[rows, :]
            pb = psel_ref[rows, :]
            acc = jnp.zeros((CHUNK, HH), jnp.float32)
            for ei in range(E_PER):
                ge = me * E_PER + ei
                coeff = jnp.where(rb == ge, pb, 0.0)
                xm = (xb * coeff).astype(jnp.bfloat16)
                acc = acc + jnp.dot(xm, w_ref[ei, :, lo:lo + HH],
                                    preferred_element_type=jnp.float32)
            return acc

        own_rows = pl.ds(me * CHUNK, CHUNK)
        xown = x_ref[own_rows, :]
        shared_out = jnp.dot(xown, shared_ref[:, :],
                             preferred_element_type=jnp.float32)

        commf_ref[0] = local_contrib(fsched_ref[0], 0).astype(jnp.bfloat16)
        commb_ref[0] = local_contrib(bsched_ref[0], HH).astype(jnp.bfloat16)

        barrier = pltpu.get_barrier_semaphore()
        for nbr in (lft, rgt):
            pl.semaphore_signal(barrier, inc=1, device_id=(nbr,),
                                device_id_type=pl.DeviceIdType.MESH)
        pl.semaphore_wait(barrier, 2)

        for s in range(N_STEP):
            rdma_f = pltpu.make_async_remote_copy(
                src_ref=commf_ref.at[s],
                dst_ref=commf_ref.at[s + 1],
                send_sem=sendf_sems.at[s],
                recv_sem=recvf_sems.at[s],
                device_id=(rgt,),
                device_id_type=pl.DeviceIdType.MESH,
            )
            rdma_b = pltpu.make_async_remote_copy(
                src_ref=commb_ref.at[s],
                dst_ref=commb_ref.at[s + 1],
                send_sem=sendb_sems.at[s],
                recv_sem=recvb_sems.at[s],
                device_id=(lft,),
                device_id_type=pl.DeviceIdType.MESH,
            )
            rdma_f.start()
            rdma_b.start()
            contrib_f = local_contrib(fsched_ref[s + 1], 0)
            contrib_b = local_contrib(bsched_ref[s + 1], HH)
            rdma_f.wait()
            rdma_b.wait()
            commf_ref[s + 1] = (
                commf_ref[s + 1].astype(jnp.float32) + contrib_f
            ).astype(jnp.bfloat16)
            commb_ref[s + 1] = (
                commb_ref[s + 1].astype(jnp.float32) + contrib_b
            ).astype(jnp.bfloat16)

        out_ref[:, 0:HH] = (
            commf_ref[N_STEP].astype(jnp.float32) + shared_out[:, 0:HH])
        out_ref[:, HH:H] = (
            commb_ref[N_STEP].astype(jnp.float32) + shared_out[:, HH:H])

    return pl.pallas_call(
        body,
        out_shape=jax.ShapeDtypeStruct((CHUNK, H), jnp.float32),
        in_specs=(
            [pl.BlockSpec(memory_space=pltpu.VMEM)] * 5
            + [pl.BlockSpec(memory_space=pltpu.SMEM)] * 4
        ),
        out_specs=pl.BlockSpec(memory_space=pltpu.VMEM),
        scratch_shapes=[
            pltpu.VMEM((N_TOK, 1), jnp.float32),
            pltpu.VMEM((N_DEV, CHUNK, HH), jnp.bfloat16),
            pltpu.VMEM((N_DEV, CHUNK, HH), jnp.bfloat16),
            pltpu.SemaphoreType.DMA((N_STEP,)),
            pltpu.SemaphoreType.DMA((N_STEP,)),
            pltpu.SemaphoreType.DMA((N_STEP,)),
            pltpu.SemaphoreType.DMA((N_STEP,)),
        ],
        compiler_params=pltpu.CompilerParams(collective_id=0),
    )(x, router_W, route_idx, expert_W, shared_W,
      right, left, fwd_sched, bwd_sched)
